# baseline (speedup 1.0000x reference)
import functools

import jax
import jax.numpy as jnp
from jax import lax
from jax.experimental import pallas as pl
from jax.experimental.pallas import tpu as pltpu
from jax.experimental.pallas import tpu_sc as plsc

B = 16384
EMBED_DIM = 128
HIDDEN = 2048

NUM_CORES = 2
NUM_SUBCORES = 16
NW = NUM_CORES * NUM_SUBCORES

BG = 4 * B
BPW = BG // NW
CHUNK = 256
NCH = BPW // CHUNK


def _sc_gather(table, idx):
    mesh = plsc.VectorSubcoreMesh(core_axis_name="c", subcore_axis_name="s")

    @functools.partial(
        pl.kernel,
        mesh=mesh,
        out_type=jax.ShapeDtypeStruct((BG, EMBED_DIM), jnp.float32),
        scratch_types=[
            pltpu.VMEM((BPW,), jnp.int32),
            pltpu.VMEM((CHUNK, EMBED_DIM), jnp.float32),
            pltpu.VMEM((CHUNK, EMBED_DIM), jnp.float32),
            pltpu.SemaphoreType.DMA,
            pltpu.SemaphoreType.DMA,
        ],
    )
    def k(table_hbm, idx_hbm, out_hbm, idx_v, rows0, rows1, sem0, sem1):
        wid = lax.axis_index("s") * NUM_CORES + lax.axis_index("c")
        base = wid * BPW
        pltpu.sync_copy(idx_hbm.at[pl.ds(base, BPW)], idx_v)
        bufs = (rows0, rows1)
        sems = (sem0, sem1)
        copies = [None, None]
        for c in range(NCH):
            s = c % 2
            copies[s] = pltpu.async_copy(
                table_hbm.at[idx_v.at[pl.ds(c * CHUNK, CHUNK)]], bufs[s], sems[s]
            )
            if c >= 1:
                p = (c - 1) % 2
                copies[p].wait()
                pltpu.sync_copy(bufs[p], out_hbm.at[pl.ds(base + (c - 1) * CHUNK, CHUNK)])
        last = (NCH - 1) % 2
        copies[last].wait()
        pltpu.sync_copy(bufs[last], out_hbm.at[pl.ds(base + (NCH - 1) * CHUNK, CHUNK)])

    return k(table, idx)


TB = 1024


def _mlp_body(g_ref, w0_ref, b0_ref, w1_ref, b1_ref, w2_ref, b2_ref,
              wout_ref, bout_ref, y_ref):
    bf = jnp.bfloat16
    h = jnp.dot(g_ref[...].astype(bf), w0_ref[...],
                preferred_element_type=jnp.float32)
    h = jnp.maximum(h + b0_ref[...], 0.0).astype(bf)
    h = jnp.dot(h, w1_ref[...], preferred_element_type=jnp.float32)
    h = jnp.maximum(h + b1_ref[...], 0.0).astype(bf)
    h = jnp.dot(h, w2_ref[...], preferred_element_type=jnp.float32)
    h = jnp.maximum(h + b2_ref[...], 0.0).astype(bf)
    y = jnp.dot(h, wout_ref[...], preferred_element_type=jnp.float32)
    y_ref[...] = y + bout_ref[...]


def _mlp(g, W0, b0, W1, b1, W2, b2, Wout, bout):
    nb = B // TB
    full = lambda shape: pl.BlockSpec(shape, lambda i: (0, 0))
    return pl.pallas_call(
        _mlp_body,
        grid=(nb,),
        in_specs=[
            pl.BlockSpec((TB, 4 * EMBED_DIM), lambda i: (i, 0)),
            full((4 * EMBED_DIM, HIDDEN)),
            full((1, HIDDEN)),
            full((HIDDEN, HIDDEN)),
            full((1, HIDDEN)),
            full((HIDDEN, HIDDEN)),
            full((1, HIDDEN)),
            full((HIDDEN, 1)),
            full((1, 1)),
        ],
        out_specs=pl.BlockSpec((TB, 1), lambda i: (i, 0)),
        out_shape=jax.ShapeDtypeStruct((B, 1), jnp.float32),
        compiler_params=pltpu.CompilerParams(
            dimension_semantics=("arbitrary",),
        ),
    )(g, W0.astype(jnp.bfloat16), b0.reshape(1, HIDDEN),
      W1.astype(jnp.bfloat16), b1.reshape(1, HIDDEN),
      W2.astype(jnp.bfloat16), b2.reshape(1, HIDDEN),
      Wout.astype(jnp.bfloat16), bout.reshape(1, 1))


def kernel(x, aug_table, mag_table, W0, b0, W1, b1, W2, b2, Wout, bout):
    del mag_table
    idx = x.reshape(-1).astype(jnp.int32)
    g = _sc_gather(aug_table, idx)
    g = g.reshape(B, 4 * EMBED_DIM)
    return _mlp(g, W0, b0, W1, b1, W2, b2, Wout, bout)


# speedup vs baseline: 1.0878x; 1.0878x over previous
import functools

import jax
import jax.numpy as jnp
from jax import lax
from jax.experimental import pallas as pl
from jax.experimental.pallas import tpu as pltpu
from jax.experimental.pallas import tpu_sc as plsc

B = 16384
EMBED_DIM = 128
HIDDEN = 2048

NUM_CORES = 2
NUM_SUBCORES = 16
NW = NUM_CORES * NUM_SUBCORES

BPW = B // NW
CHUNK = 256
NCH = BPW // CHUNK
NIT = 4 * NCH


def _sc_gather(table, xt):
    mesh = plsc.VectorSubcoreMesh(core_axis_name="c", subcore_axis_name="s")

    @functools.partial(
        pl.kernel,
        mesh=mesh,
        out_type=jax.ShapeDtypeStruct((4, B, EMBED_DIM), jnp.float32),
        scratch_types=[
            pltpu.VMEM((4 * BPW,), jnp.int32),
            pltpu.VMEM((CHUNK, EMBED_DIM), jnp.float32),
            pltpu.VMEM((CHUNK, EMBED_DIM), jnp.float32),
            pltpu.SemaphoreType.DMA,
            pltpu.SemaphoreType.DMA,
            pltpu.SemaphoreType.DMA,
            pltpu.SemaphoreType.DMA,
        ],
    )
    def k(table_hbm, xt_hbm, out_hbm, idx_v, rows0, rows1, g0, g1, s0, s1):
        wid = lax.axis_index("s") * NUM_CORES + lax.axis_index("c")
        base = wid * BPW
        for c in range(4):
            pltpu.sync_copy(xt_hbm.at[c, pl.ds(base, BPW)],
                            idx_v.at[pl.ds(c * BPW, BPW)])
        bufs = (rows0, rows1)
        gsems = (g0, g1)
        ssems = (s0, s1)
        gathers = [None, None]
        scatters = [None, None]
        for it in range(NIT):
            s = it % 2
            c, j = divmod(it, NCH)
            if it >= 2:
                scatters[s].wait()
            gathers[s] = pltpu.async_copy(
                table_hbm.at[idx_v.at[pl.ds(it * CHUNK, CHUNK)]],
                bufs[s], gsems[s])
            if it >= 1:
                p = (it - 1) % 2
                pc, pj = divmod(it - 1, NCH)
                gathers[p].wait()
                scatters[p] = pltpu.async_copy(
                    bufs[p],
                    out_hbm.at[pc, pl.ds(base + pj * CHUNK, CHUNK)],
                    ssems[p])
        last = (NIT - 1) % 2
        lc, lj = divmod(NIT - 1, NCH)
        gathers[last].wait()
        scatters[last] = pltpu.async_copy(
            bufs[last],
            out_hbm.at[lc, pl.ds(base + lj * CHUNK, CHUNK)],
            ssems[last])
        scatters[(NIT - 2) % 2].wait()
        scatters[last].wait()

    return k(table, xt)


TB = 1024


def _mlp_body(g_ref, w0_ref, b0_ref, w1_ref, b1_ref, w2_ref, b2_ref,
              wout_ref, bout_ref, y_ref):
    bf = jnp.bfloat16
    g = jnp.concatenate(
        [g_ref[0], g_ref[1], g_ref[2], g_ref[3]], axis=1).astype(bf)
    h = jnp.dot(g, w0_ref[...], preferred_element_type=jnp.float32)
    h = jnp.maximum(h + b0_ref[...], 0.0).astype(bf)
    h = jnp.dot(h, w1_ref[...], preferred_element_type=jnp.float32)
    h = jnp.maximum(h + b1_ref[...], 0.0).astype(bf)
    h = jnp.dot(h, w2_ref[...], preferred_element_type=jnp.float32)
    h = jnp.maximum(h + b2_ref[...], 0.0).astype(bf)
    y = jnp.dot(h, wout_ref[...], preferred_element_type=jnp.float32)
    y_ref[...] = y + bout_ref[...]


def _mlp(g4, W0, b0, W1, b1, W2, b2, Wout, bout):
    nb = B // TB
    full = lambda shape: pl.BlockSpec(shape, lambda i: (0, 0))
    return pl.pallas_call(
        _mlp_body,
        grid=(nb,),
        in_specs=[
            pl.BlockSpec((4, TB, EMBED_DIM), lambda i: (0, i, 0)),
            full((4 * EMBED_DIM, HIDDEN)),
            full((1, HIDDEN)),
            full((HIDDEN, HIDDEN)),
            full((1, HIDDEN)),
            full((HIDDEN, HIDDEN)),
            full((1, HIDDEN)),
            full((HIDDEN, 1)),
            full((1, 1)),
        ],
        out_specs=pl.BlockSpec((TB, 1), lambda i: (i, 0)),
        out_shape=jax.ShapeDtypeStruct((B, 1), jnp.float32),
        compiler_params=pltpu.CompilerParams(
            dimension_semantics=("arbitrary",),
        ),
    )(g4, W0.astype(jnp.bfloat16), b0.reshape(1, HIDDEN),
      W1.astype(jnp.bfloat16), b1.reshape(1, HIDDEN),
      W2.astype(jnp.bfloat16), b2.reshape(1, HIDDEN),
      Wout.astype(jnp.bfloat16), bout.reshape(1, 1))


def kernel(x, aug_table, mag_table, W0, b0, W1, b1, W2, b2, Wout, bout):
    del mag_table
    xt = x.T.astype(jnp.int32)
    g4 = _sc_gather(aug_table, xt)
    return _mlp(g4, W0, b0, W1, b1, W2, b2, Wout, bout)


# speedup vs baseline: 1.1323x; 1.0409x over previous
import functools

import jax
import jax.numpy as jnp
from jax import lax
from jax.experimental import pallas as pl
from jax.experimental.pallas import tpu as pltpu
from jax.experimental.pallas import tpu_sc as plsc

B = 16384
EMBED_DIM = 128
HIDDEN = 2048

NC = 4
BC = B // NC

NUM_CORES = 2
NUM_SUBCORES = 16
NW = NUM_CORES * NUM_SUBCORES

BPW = BC // NW
NIT = 4


def _sc_gather(table, xt):
    mesh = plsc.VectorSubcoreMesh(core_axis_name="c", subcore_axis_name="s")

    @functools.partial(
        pl.kernel,
        mesh=mesh,
        out_type=jax.ShapeDtypeStruct((4, BC, EMBED_DIM), jnp.float32),
        scratch_types=[
            pltpu.VMEM((4 * BPW,), jnp.int32),
            pltpu.VMEM((BPW, EMBED_DIM), jnp.float32),
            pltpu.VMEM((BPW, EMBED_DIM), jnp.float32),
            pltpu.SemaphoreType.DMA,
            pltpu.SemaphoreType.DMA,
            pltpu.SemaphoreType.DMA,
            pltpu.SemaphoreType.DMA,
        ],
    )
    def k(table_hbm, xt_hbm, out_hbm, idx_v, rows0, rows1, g0, g1, s0, s1):
        wid = lax.axis_index("s") * NUM_CORES + lax.axis_index("c")
        base = wid * BPW
        for c in range(4):
            pltpu.sync_copy(xt_hbm.at[c, pl.ds(base, BPW)],
                            idx_v.at[pl.ds(c * BPW, BPW)])
        bufs = (rows0, rows1)
        gsems = (g0, g1)
        ssems = (s0, s1)
        gathers = [None, None]
        scatters = [None, None]
        for it in range(NIT):
            s = it % 2
            if it >= 2:
                scatters[s].wait()
            gathers[s] = pltpu.async_copy(
                table_hbm.at[idx_v.at[pl.ds(it * BPW, BPW)]],
                bufs[s], gsems[s])
            if it >= 1:
                p = (it - 1) % 2
                gathers[p].wait()
                scatters[p] = pltpu.async_copy(
                    bufs[p], out_hbm.at[it - 1, pl.ds(base, BPW)], ssems[p])
        last = (NIT - 1) % 2
        gathers[last].wait()
        scatters[last] = pltpu.async_copy(
            bufs[last], out_hbm.at[NIT - 1, pl.ds(base, BPW)], ssems[last])
        scatters[(NIT - 2) % 2].wait()
        scatters[last].wait()

    return k(table, xt)


TB = 1024


def _mlp_body(g_ref, w0_ref, b0_ref, w1_ref, b1_ref, w2_ref, b2_ref,
              wout_ref, bout_ref, y_ref):
    bf = jnp.bfloat16
    g = jnp.concatenate(
        [g_ref[0], g_ref[1], g_ref[2], g_ref[3]], axis=1).astype(bf)
    h = jnp.dot(g, w0_ref[...], preferred_element_type=jnp.float32)
    h = jnp.maximum(h + b0_ref[...], 0.0).astype(bf)
    h = jnp.dot(h, w1_ref[...], preferred_element_type=jnp.float32)
    h = jnp.maximum(h + b1_ref[...], 0.0).astype(bf)
    h = jnp.dot(h, w2_ref[...], preferred_element_type=jnp.float32)
    h = jnp.maximum(h + b2_ref[...], 0.0).astype(bf)
    y = jnp.dot(h, wout_ref[...], preferred_element_type=jnp.float32)
    y_ref[...] = y + bout_ref[...]


def _mlp(g4, W0b, b0, W1b, b1, W2b, b2, Woutb, bout):
    nb = BC // TB
    full = lambda shape: pl.BlockSpec(shape, lambda i: (0, 0))
    return pl.pallas_call(
        _mlp_body,
        grid=(nb,),
        in_specs=[
            pl.BlockSpec((4, TB, EMBED_DIM), lambda i: (0, i, 0)),
            full((4 * EMBED_DIM, HIDDEN)),
            full((1, HIDDEN)),
            full((HIDDEN, HIDDEN)),
            full((1, HIDDEN)),
            full((HIDDEN, HIDDEN)),
            full((1, HIDDEN)),
            full((HIDDEN, 1)),
            full((1, 1)),
        ],
        out_specs=pl.BlockSpec((TB, 1), lambda i: (i, 0)),
        out_shape=jax.ShapeDtypeStruct((BC, 1), jnp.float32),
        compiler_params=pltpu.CompilerParams(
            dimension_semantics=("arbitrary",),
        ),
    )(g4, W0b, b0, W1b, b1, W2b, b2, Woutb, bout)


def kernel(x, aug_table, mag_table, W0, b0, W1, b1, W2, b2, Wout, bout):
    del mag_table
    xt = x.T.astype(jnp.int32)
    bf = jnp.bfloat16
    W0b, W1b, W2b, Woutb = (W0.astype(bf), W1.astype(bf), W2.astype(bf),
                            Wout.astype(bf))
    b0r = b0.reshape(1, HIDDEN)
    b1r = b1.reshape(1, HIDDEN)
    b2r = b2.reshape(1, HIDDEN)
    boutr = bout.reshape(1, 1)
    gs = [_sc_gather(aug_table, xt[:, k * BC:(k + 1) * BC]) for k in range(NC)]
    ys = [_mlp(g, W0b, b0r, W1b, b1r, W2b, b2r, Woutb, boutr) for g in gs]
    return jnp.concatenate(ys, axis=0)


# speedup vs baseline: 1.1534x; 1.0186x over previous
import functools

import jax
import jax.numpy as jnp
from jax import lax
from jax.experimental import pallas as pl
from jax.experimental.pallas import tpu as pltpu
from jax.experimental.pallas import tpu_sc as plsc

B = 16384
EMBED_DIM = 128
HIDDEN = 2048

NC = 2
BC = B // NC

NUM_CORES = 2
NUM_SUBCORES = 16
NW = NUM_CORES * NUM_SUBCORES

BPW = BC // NW
NIT = 4


def _sc_gather(table, xt):
    mesh = plsc.VectorSubcoreMesh(core_axis_name="c", subcore_axis_name="s")

    @functools.partial(
        pl.kernel,
        mesh=mesh,
        out_type=jax.ShapeDtypeStruct((4, BC, EMBED_DIM), jnp.float32),
        scratch_types=[
            pltpu.VMEM((4 * BPW,), jnp.int32),
            pltpu.VMEM((BPW, EMBED_DIM), jnp.float32),
            pltpu.VMEM((BPW, EMBED_DIM), jnp.float32),
            pltpu.SemaphoreType.DMA,
            pltpu.SemaphoreType.DMA,
            pltpu.SemaphoreType.DMA,
            pltpu.SemaphoreType.DMA,
        ],
    )
    def k(table_hbm, xt_hbm, out_hbm, idx_v, rows0, rows1, g0, g1, s0, s1):
        wid = lax.axis_index("s") * NUM_CORES + lax.axis_index("c")
        base = wid * BPW
        for c in range(4):
            pltpu.sync_copy(xt_hbm.at[c, pl.ds(base, BPW)],
                            idx_v.at[pl.ds(c * BPW, BPW)])
        bufs = (rows0, rows1)
        gsems = (g0, g1)
        ssems = (s0, s1)
        gathers = [None, None]
        scatters = [None, None]
        for it in range(NIT):
            s = it % 2
            if it >= 2:
                scatters[s].wait()
            gathers[s] = pltpu.async_copy(
                table_hbm.at[idx_v.at[pl.ds(it * BPW, BPW)]],
                bufs[s], gsems[s])
            if it >= 1:
                p = (it - 1) % 2
                gathers[p].wait()
                scatters[p] = pltpu.async_copy(
                    bufs[p], out_hbm.at[it - 1, pl.ds(base, BPW)], ssems[p])
        last = (NIT - 1) % 2
        gathers[last].wait()
        scatters[last] = pltpu.async_copy(
            bufs[last], out_hbm.at[NIT - 1, pl.ds(base, BPW)], ssems[last])
        scatters[(NIT - 2) % 2].wait()
        scatters[last].wait()

    return k(table, xt)


TB = 1024


def _mlp_body(g_ref, w0_ref, b0_ref, w1_ref, b1_ref, w2_ref, b2_ref,
              wout_ref, bout_ref, y_ref):
    bf = jnp.bfloat16
    g = jnp.concatenate(
        [g_ref[0], g_ref[1], g_ref[2], g_ref[3]], axis=1).astype(bf)
    h = jnp.dot(g, w0_ref[...], preferred_element_type=jnp.float32)
    h = jnp.maximum(h + b0_ref[...], 0.0).astype(bf)
    h = jnp.dot(h, w1_ref[...], preferred_element_type=jnp.float32)
    h = jnp.maximum(h + b1_ref[...], 0.0).astype(bf)
    h = jnp.dot(h, w2_ref[...], preferred_element_type=jnp.float32)
    h = jnp.maximum(h + b2_ref[...], 0.0).astype(bf)
    y = jnp.dot(h, wout_ref[...], preferred_element_type=jnp.float32)
    y_ref[...] = y + bout_ref[...]


def _mlp(g4, W0b, b0, W1b, b1, W2b, b2, Woutb, bout):
    nb = BC // TB
    full = lambda shape: pl.BlockSpec(shape, lambda i: (0, 0))
    return pl.pallas_call(
        _mlp_body,
        grid=(nb,),
        in_specs=[
            pl.BlockSpec((4, TB, EMBED_DIM), lambda i: (0, i, 0)),
            full((4 * EMBED_DIM, HIDDEN)),
            full((1, HIDDEN)),
            full((HIDDEN, HIDDEN)),
            full((1, HIDDEN)),
            full((HIDDEN, HIDDEN)),
            full((1, HIDDEN)),
            full((HIDDEN, 1)),
            full((1, 1)),
        ],
        out_specs=pl.BlockSpec((TB, 1), lambda i: (i, 0)),
        out_shape=jax.ShapeDtypeStruct((BC, 1), jnp.float32),
        compiler_params=pltpu.CompilerParams(
            dimension_semantics=("arbitrary",),
        ),
    )(g4, W0b, b0, W1b, b1, W2b, b2, Woutb, bout)


def kernel(x, aug_table, mag_table, W0, b0, W1, b1, W2, b2, Wout, bout):
    del mag_table
    xt = x.T.astype(jnp.int32)
    bf = jnp.bfloat16
    W0b, W1b, W2b, Woutb = (W0.astype(bf), W1.astype(bf), W2.astype(bf),
                            Wout.astype(bf))
    b0r = b0.reshape(1, HIDDEN)
    b1r = b1.reshape(1, HIDDEN)
    b2r = b2.reshape(1, HIDDEN)
    boutr = bout.reshape(1, 1)
    gs = [_sc_gather(aug_table, xt[:, k * BC:(k + 1) * BC]) for k in range(NC)]
    ys = [_mlp(g, W0b, b0r, W1b, b1r, W2b, b2r, Woutb, boutr) for g in gs]
    return jnp.concatenate(ys, axis=0)
